# baseline (speedup 1.0000x reference)
import functools

import jax
import jax.numpy as jnp
from jax.experimental import pallas as pl
from jax.experimental.pallas import tpu as pltpu
import jax.experimental.pallas.tpu_sc as plsc

T = 2048
D = 768
F = 3072
E = 8
EPS = 1e-6

TM = 256
NF = 4
TF = F // NF
M_TILES = T // TM + E
PADDED = M_TILES * TM

GW_PAD = 128

_NC = 2
_NS = 16
_LANES = 16
_CHUNK = T // _NS
_NV = _CHUNK // _LANES
_HALF = _CHUNK // 2
_GCHUNK = T // (_NC * _NS)


def _gate_body(x_ref, nw_ref, gw_ref, xn_ref, sco_ref, idx_ref):
    xb = x_ref[...]
    ms = jnp.mean(xb * xb, axis=1, keepdims=True)
    xn = xb * jax.lax.rsqrt(ms + EPS) * nw_ref[...]
    xn_ref[...] = xn
    logits = jnp.dot(xn.astype(jnp.bfloat16),
                     gw_ref[...].astype(jnp.bfloat16),
                     preferred_element_type=jnp.float32)
    col = jax.lax.broadcasted_iota(jnp.int32, logits.shape, 1)
    logits = jnp.where(col < E, logits, -1e30)
    mx = jnp.max(logits, axis=1, keepdims=True)
    idx_ref[...] = jnp.min(jnp.where(logits >= mx, col, E), axis=1,
                           keepdims=True)
    ssum = jnp.sum(jnp.exp(logits - mx), axis=1, keepdims=True)
    sco_ref[...] = 1.0 / ssum


def _route_body(idx_hbm, sco_hbm, xn_hbm,
                xs_hbm, ss_hbm, pos_hbm, te_hbm, tv_hbm,
                idx_v, sco_v, pos_v, pos_lo, pos_hi, rows_v,
                cnt_v, all_v, te_v, tv_v,
                counts_sh, sem):
    cid = jax.lax.axis_index("c")
    sid = jax.lax.axis_index("s")
    tok0 = sid * _CHUNK
    lane = jax.lax.iota(jnp.int32, _LANES)
    zeros = jnp.zeros((_LANES,), jnp.int32)
    ones = jnp.ones((_LANES,), jnp.int32)

    pltpu.sync_copy(idx_hbm.at[pl.ds(tok0, _CHUNK)], idx_v)
    cnt = zeros
    for v in range(_NV):
        xv = idx_v[pl.ds(v * _LANES, _LANES)]
        for e in range(E):
            c = plsc.all_reduce_population_count(xv == e)
            cnt = jnp.where(lane == e, cnt + c, cnt)
    cnt_v[...] = cnt
    pltpu.sync_copy(cnt_v, counts_sh.at[pl.ds(sid * _LANES, _LANES)])
    plsc.subcore_barrier()

    pltpu.sync_copy(counts_sh, all_v)
    tot = zeros
    pre = zeros
    svec = jnp.full((_LANES,), sid, jnp.int32)
    for t in range(_NS):
        ct = all_v[pl.ds(t * _LANES, _LANES)]
        tvec = jnp.full((_LANES,), t, jnp.int32)
        pre = jnp.where(tvec < svec, pre + ct, pre)
        tot = tot + ct
    al = ((tot + (TM - 1)) >> 8) << 8
    cs = plsc.cumsum(al)
    start = cs - al
    base = start + pre

    b = [jnp.full((_LANES,), base[e], jnp.int32) for e in range(E)]
    for v in range(_NV):
        xv = idx_v[pl.ds(v * _LANES, _LANES)]
        pos = zeros
        for e in range(E):
            m = xv == e
            incl = plsc.cumsum(jnp.where(m, ones, zeros))
            pos = jnp.where(m, b[e] + incl - 1, pos)
            b[e] = b[e] + plsc.all_reduce_population_count(m)
        pos_v[pl.ds(v * _LANES, _LANES)] = pos
        if v < _NV // 2:
            pos_lo[pl.ds(v * _LANES, _LANES)] = pos
        else:
            pos_hi[pl.ds((v - _NV // 2) * _LANES, _LANES)] = pos

    @pl.when(cid == 0)
    def _():
        pltpu.sync_copy(pos_v, pos_hbm.at[pl.ds(tok0, _CHUNK)])
        pltpu.sync_copy(sco_hbm.at[pl.ds(tok0, _CHUNK)], sco_v)
        pltpu.async_copy(sco_v, ss_hbm.at[pos_v], sem).wait()

    @pl.when(cid == 0)
    def _():
        pltpu.sync_copy(xn_hbm.at[pl.ds(tok0, _HALF)], rows_v)
        pltpu.async_copy(rows_v, xs_hbm.at[pos_lo], sem).wait()

    @pl.when(cid == 1)
    def _():
        pltpu.sync_copy(xn_hbm.at[pl.ds(tok0 + _HALF, _HALF)], rows_v)
        pltpu.async_copy(rows_v, xs_hbm.at[pos_hi], sem).wait()

    @pl.when((cid == 0) & (sid == 0))
    def _():
        e_last = zeros
        for e in range(E):
            e_tot = jnp.full((_LANES,), tot[e], jnp.int32)
            e_vec = jnp.full((_LANES,), e, jnp.int32)
            e_last = jnp.where(e_tot > 0, e_vec, e_last)
        m16 = lane * TM
        te = e_last
        tv = zeros
        for e in range(E):
            s_e = jnp.full((_LANES,), start[e], jnp.int32)
            a_e = jnp.full((_LANES,), al[e], jnp.int32)
            in_r = (m16 >= s_e) & (m16 < s_e + a_e)
            te = jnp.where(in_r, jnp.full((_LANES,), e, jnp.int32), te)
            tv = jnp.where(in_r, ones, tv)
        te_v[...] = te
        tv_v[...] = tv
        pltpu.sync_copy(te_v, te_hbm)
        pltpu.sync_copy(tv_v, tv_hbm)


def _ffn_body(te_ref, tv_ref, x_ref, w1_ref, b1_ref, w2_ref, b2_ref, ss_ref,
              out_ref):
    m = pl.program_id(0)

    @pl.when(tv_ref[m] != 0)
    def _():
        h = jnp.dot(x_ref[...].astype(jnp.bfloat16), w1_ref[0],
                    preferred_element_type=jnp.float32) + b1_ref[0]
        h = h * (1.0 / (1.0 + jnp.exp(-h)))
        y = jnp.dot(h.astype(jnp.bfloat16), w2_ref[0],
                    preferred_element_type=jnp.float32)
        out_ref[...] = (y + b2_ref[0]) * ss_ref[...]


def _unsort_body(pos_hbm, ys_hbm, out_hbm, pidx, rows, sem):
    cid = jax.lax.axis_index("c")
    sid = jax.lax.axis_index("s")
    t0 = (sid * _NC + cid) * _GCHUNK
    pltpu.sync_copy(pos_hbm.at[pl.ds(t0, _GCHUNK)], pidx)
    pltpu.async_copy(ys_hbm.at[pidx], rows, sem).wait()
    pltpu.sync_copy(rows, out_hbm.at[pl.ds(t0, _GCHUNK)])


def _sel_f(tv, f):
    return jnp.where(tv != 0, f, 0)


@jax.jit
def kernel(x, norm_w, gate_w, W1, b1, W2, b2):
    x2 = x.reshape(T, D)
    nw2 = norm_w.reshape(1, D)
    gw_p = jnp.zeros((D, GW_PAD), jnp.float32).at[:, :E].set(gate_w)

    xn, sco2, idx2 = pl.pallas_call(
        _gate_body,
        grid=(T // TM,),
        in_specs=[
            pl.BlockSpec((TM, D), lambda m: (m, 0)),
            pl.BlockSpec((1, D), lambda m: (0, 0)),
            pl.BlockSpec((D, GW_PAD), lambda m: (0, 0)),
        ],
        out_specs=[
            pl.BlockSpec((TM, D), lambda m: (m, 0)),
            pl.BlockSpec((TM, 1), lambda m: (m, 0)),
            pl.BlockSpec((TM, 1), lambda m: (m, 0)),
        ],
        out_shape=[
            jax.ShapeDtypeStruct((T, D), jnp.float32),
            jax.ShapeDtypeStruct((T, 1), jnp.float32),
            jax.ShapeDtypeStruct((T, 1), jnp.int32),
        ],
        compiler_params=pltpu.CompilerParams(
            dimension_semantics=("arbitrary",)),
    )(x2, nw2, gw_p)
    idx1 = idx2.reshape(T)
    sco1 = sco2.reshape(T)

    mesh = plsc.VectorSubcoreMesh(core_axis_name="c", subcore_axis_name="s")
    route = pl.kernel(
        _route_body,
        out_type=(
            jax.ShapeDtypeStruct((PADDED, D), jnp.float32),
            jax.ShapeDtypeStruct((PADDED,), jnp.float32),
            jax.ShapeDtypeStruct((T,), jnp.int32),
            jax.ShapeDtypeStruct((_LANES,), jnp.int32),
            jax.ShapeDtypeStruct((_LANES,), jnp.int32),
        ),
        mesh=mesh,
        scratch_types=(
            pltpu.VMEM((_CHUNK,), jnp.int32),
            pltpu.VMEM((_CHUNK,), jnp.float32),
            pltpu.VMEM((_CHUNK,), jnp.int32),
            pltpu.VMEM((_HALF,), jnp.int32),
            pltpu.VMEM((_HALF,), jnp.int32),
            pltpu.VMEM((_HALF, D), jnp.float32),
            pltpu.VMEM((_LANES,), jnp.int32),
            pltpu.VMEM((_NS * _LANES,), jnp.int32),
            pltpu.VMEM((_LANES,), jnp.int32),
            pltpu.VMEM((_LANES,), jnp.int32),
            pltpu.VMEM_SHARED((_NS * _LANES,), jnp.int32),
            pltpu.SemaphoreType.DMA,
        ),
        compiler_params=pltpu.CompilerParams(needs_layout_passes=False),
    )
    xs, ss, pos, te, tv = route(idx1, sco1, xn)
    ss2 = ss.reshape(PADDED, 1)
    b1r = b1.reshape(E, 1, F)
    b2r = b2.reshape(E, 1, D)
    W1b = W1.astype(jnp.bfloat16)
    W2b = W2.astype(jnp.bfloat16)

    ys = pl.pallas_call(
        _ffn_body,
        grid_spec=pltpu.PrefetchScalarGridSpec(
            num_scalar_prefetch=2,
            grid=(M_TILES,),
            in_specs=[
                pl.BlockSpec((TM, D),
                             lambda m, te, tv: (_sel_f(tv[m], m), 0)),
                pl.BlockSpec((1, D, F), lambda m, te, tv: (te[m], 0, 0)),
                pl.BlockSpec((1, 1, F), lambda m, te, tv: (te[m], 0, 0)),
                pl.BlockSpec((1, F, D), lambda m, te, tv: (te[m], 0, 0)),
                pl.BlockSpec((1, 1, D), lambda m, te, tv: (te[m], 0, 0)),
                pl.BlockSpec((TM, 1),
                             lambda m, te, tv: (_sel_f(tv[m], m), 0)),
            ],
            out_specs=pl.BlockSpec((TM, D), lambda m, te, tv: (m, 0)),
        ),
        out_shape=jax.ShapeDtypeStruct((PADDED, D), jnp.float32),
        compiler_params=pltpu.CompilerParams(
            dimension_semantics=("arbitrary",)),
    )(te, tv, xs, W1b, b1r, W2b, b2r, ss2)

    unsort = pl.kernel(
        _unsort_body,
        out_type=jax.ShapeDtypeStruct((T, D), jnp.float32),
        mesh=plsc.VectorSubcoreMesh(core_axis_name="c", subcore_axis_name="s"),
        scratch_types=(
            pltpu.VMEM((_GCHUNK,), jnp.int32),
            pltpu.VMEM((_GCHUNK, D), jnp.float32),
            pltpu.SemaphoreType.DMA,
        ),
    )
    out = unsort(pos, ys)
    return out.reshape(1, T, D)


# speedup vs baseline: 1.0420x; 1.0420x over previous
import functools

import jax
import jax.numpy as jnp
from jax.experimental import pallas as pl
from jax.experimental.pallas import tpu as pltpu
import jax.experimental.pallas.tpu_sc as plsc

T = 2048
D = 768
F = 3072
E = 8
EPS = 1e-6

TM = 256
NF = 4
TF = F // NF
M_TILES = T // TM + E
PADDED = M_TILES * TM

GW_PAD = 128

_NC = 2
_NS = 16
_LANES = 16
_CHUNK = T // _NS
_NV = _CHUNK // _LANES
_HALF = _CHUNK // 2
_GCHUNK = T // (_NC * _NS)


def _gate_body(x_ref, nw_ref, gw_ref, xn_ref, sco_ref, idx_ref):
    xb = x_ref[...]
    ms = jnp.mean(xb * xb, axis=1, keepdims=True)
    xn = xb * jax.lax.rsqrt(ms + EPS) * nw_ref[...]
    xn_ref[...] = xn
    logits = jnp.dot(xn.astype(jnp.bfloat16),
                     gw_ref[...].astype(jnp.bfloat16),
                     preferred_element_type=jnp.float32)
    col = jax.lax.broadcasted_iota(jnp.int32, logits.shape, 1)
    logits = jnp.where(col < E, logits, -1e30)
    mx = jnp.max(logits, axis=1, keepdims=True)
    idx_ref[...] = jnp.min(jnp.where(logits >= mx, col, E), axis=1,
                           keepdims=True)
    ssum = jnp.sum(jnp.exp(logits - mx), axis=1, keepdims=True)
    sco_ref[...] = 1.0 / ssum


def _route_body(idx_hbm, sco_hbm, xn_hbm,
                xs_hbm, ss_hbm, pos_hbm, te_hbm, tv_hbm,
                idx_v, sco_v, pos_v, pos_lo, pos_hi, rows_v,
                cnt_v, all_v, te_v, tv_v,
                counts_sh, sco_sh, sem):
    cid = jax.lax.axis_index("c")
    sid = jax.lax.axis_index("s")
    tok0 = sid * _CHUNK
    lane = jax.lax.iota(jnp.int32, _LANES)
    zeros = jnp.zeros((_LANES,), jnp.int32)
    ones = jnp.ones((_LANES,), jnp.int32)

    pltpu.sync_copy(idx_hbm.at[pl.ds(tok0, _CHUNK)], idx_v)
    cnt = zeros
    for v in range(_NV):
        xv = idx_v[pl.ds(v * _LANES, _LANES)]
        for e in range(E):
            c = plsc.all_reduce_population_count(xv == e)
            cnt = jnp.where(lane == e, cnt + c, cnt)
    cnt_v[...] = cnt
    pltpu.sync_copy(cnt_v, counts_sh.at[pl.ds(sid * _LANES, _LANES)])
    plsc.subcore_barrier()

    pltpu.sync_copy(counts_sh, all_v)
    tot = zeros
    pre = zeros
    svec = jnp.full((_LANES,), sid, jnp.int32)
    for t in range(_NS):
        ct = all_v[pl.ds(t * _LANES, _LANES)]
        tvec = jnp.full((_LANES,), t, jnp.int32)
        pre = jnp.where(tvec < svec, pre + ct, pre)
        tot = tot + ct
    al = ((tot + (TM - 1)) >> 8) << 8
    cs = plsc.cumsum(al)
    start = cs - al
    base = start + pre

    b = [jnp.full((_LANES,), base[e], jnp.int32) for e in range(E)]
    for v in range(_NV):
        xv = idx_v[pl.ds(v * _LANES, _LANES)]
        pos = zeros
        for e in range(E):
            m = xv == e
            incl = plsc.cumsum(jnp.where(m, ones, zeros))
            pos = jnp.where(m, b[e] + incl - 1, pos)
            b[e] = b[e] + plsc.all_reduce_population_count(m)
        pos_v[pl.ds(v * _LANES, _LANES)] = pos
        if v < _NV // 2:
            pos_lo[pl.ds(v * _LANES, _LANES)] = pos
        else:
            pos_hi[pl.ds((v - _NV // 2) * _LANES, _LANES)] = pos

    @pl.when(cid == 0)
    def _():
        pltpu.sync_copy(pos_v, pos_hbm.at[pl.ds(tok0, _CHUNK)])
        pltpu.sync_copy(xn_hbm.at[pl.ds(tok0, _HALF)], rows_v)
        pltpu.async_copy(rows_v, xs_hbm.at[pos_lo], sem).wait()

    @pl.when(cid == 1)
    def _():
        pltpu.sync_copy(sco_hbm.at[pl.ds(tok0, _CHUNK)], sco_v)
        pltpu.sync_copy(sco_v, sco_sh.at[pos_v])
        pltpu.sync_copy(xn_hbm.at[pl.ds(tok0 + _HALF, _HALF)], rows_v)
        pltpu.async_copy(rows_v, xs_hbm.at[pos_hi], sem).wait()

    plsc.subcore_barrier()

    @pl.when((cid == 1) & (sid == 0))
    def _():
        pltpu.sync_copy(sco_sh, ss_hbm)

    @pl.when((cid == 0) & (sid == 0))
    def _():
        e_last = zeros
        for e in range(E):
            e_tot = jnp.full((_LANES,), tot[e], jnp.int32)
            e_vec = jnp.full((_LANES,), e, jnp.int32)
            e_last = jnp.where(e_tot > 0, e_vec, e_last)
        m16 = lane * TM
        te = e_last
        tv = zeros
        for e in range(E):
            s_e = jnp.full((_LANES,), start[e], jnp.int32)
            a_e = jnp.full((_LANES,), al[e], jnp.int32)
            in_r = (m16 >= s_e) & (m16 < s_e + a_e)
            te = jnp.where(in_r, jnp.full((_LANES,), e, jnp.int32), te)
            tv = jnp.where(in_r, ones, tv)
        te_v[...] = te
        tv_v[...] = tv
        pltpu.sync_copy(te_v, te_hbm)
        pltpu.sync_copy(tv_v, tv_hbm)


def _ffn_body(te_ref, tv_ref, x_ref, w1_ref, b1_ref, w2_ref, b2_ref, ss_ref,
              out_ref):
    m = pl.program_id(0)

    @pl.when(tv_ref[m] != 0)
    def _():
        h = jnp.dot(x_ref[...].astype(jnp.bfloat16), w1_ref[0],
                    preferred_element_type=jnp.float32) + b1_ref[0]
        h = h * (1.0 / (1.0 + jnp.exp(-h)))
        y = jnp.dot(h.astype(jnp.bfloat16), w2_ref[0],
                    preferred_element_type=jnp.float32)
        out_ref[...] = (y + b2_ref[0]) * ss_ref[...]


def _unsort_body(pos_hbm, ys_hbm, out_hbm, pidx, rows, sem):
    cid = jax.lax.axis_index("c")
    sid = jax.lax.axis_index("s")
    t0 = (sid * _NC + cid) * _GCHUNK
    pltpu.sync_copy(pos_hbm.at[pl.ds(t0, _GCHUNK)], pidx)
    pltpu.async_copy(ys_hbm.at[pidx], rows, sem).wait()
    pltpu.sync_copy(rows, out_hbm.at[pl.ds(t0, _GCHUNK)])


def _sel_f(tv, f):
    return jnp.where(tv != 0, f, 0)


@jax.jit
def kernel(x, norm_w, gate_w, W1, b1, W2, b2):
    x2 = x.reshape(T, D)
    nw2 = norm_w.reshape(1, D)
    gw_p = jnp.zeros((D, GW_PAD), jnp.float32).at[:, :E].set(gate_w)

    xn, sco2, idx2 = pl.pallas_call(
        _gate_body,
        grid=(T // TM,),
        in_specs=[
            pl.BlockSpec((TM, D), lambda m: (m, 0)),
            pl.BlockSpec((1, D), lambda m: (0, 0)),
            pl.BlockSpec((D, GW_PAD), lambda m: (0, 0)),
        ],
        out_specs=[
            pl.BlockSpec((TM, D), lambda m: (m, 0)),
            pl.BlockSpec((TM, 1), lambda m: (m, 0)),
            pl.BlockSpec((TM, 1), lambda m: (m, 0)),
        ],
        out_shape=[
            jax.ShapeDtypeStruct((T, D), jnp.float32),
            jax.ShapeDtypeStruct((T, 1), jnp.float32),
            jax.ShapeDtypeStruct((T, 1), jnp.int32),
        ],
        compiler_params=pltpu.CompilerParams(
            dimension_semantics=("arbitrary",)),
    )(x2, nw2, gw_p)
    idx1 = idx2.reshape(T)
    sco1 = sco2.reshape(T)

    mesh = plsc.VectorSubcoreMesh(core_axis_name="c", subcore_axis_name="s")
    route = pl.kernel(
        _route_body,
        out_type=(
            jax.ShapeDtypeStruct((PADDED, D), jnp.float32),
            jax.ShapeDtypeStruct((PADDED,), jnp.float32),
            jax.ShapeDtypeStruct((T,), jnp.int32),
            jax.ShapeDtypeStruct((_LANES,), jnp.int32),
            jax.ShapeDtypeStruct((_LANES,), jnp.int32),
        ),
        mesh=mesh,
        scratch_types=(
            pltpu.VMEM((_CHUNK,), jnp.int32),
            pltpu.VMEM((_CHUNK,), jnp.float32),
            pltpu.VMEM((_CHUNK,), jnp.int32),
            pltpu.VMEM((_HALF,), jnp.int32),
            pltpu.VMEM((_HALF,), jnp.int32),
            pltpu.VMEM((_HALF, D), jnp.float32),
            pltpu.VMEM((_LANES,), jnp.int32),
            pltpu.VMEM((_NS * _LANES,), jnp.int32),
            pltpu.VMEM((_LANES,), jnp.int32),
            pltpu.VMEM((_LANES,), jnp.int32),
            pltpu.VMEM_SHARED((_NS * _LANES,), jnp.int32),
            pltpu.VMEM_SHARED((PADDED,), jnp.float32),
            pltpu.SemaphoreType.DMA,
        ),
        compiler_params=pltpu.CompilerParams(needs_layout_passes=False),
    )
    xs, ss, pos, te, tv = route(idx1, sco1, xn)
    ss2 = ss.reshape(PADDED, 1)
    b1r = b1.reshape(E, 1, F)
    b2r = b2.reshape(E, 1, D)
    W1b = W1.astype(jnp.bfloat16)
    W2b = W2.astype(jnp.bfloat16)

    ys = pl.pallas_call(
        _ffn_body,
        grid_spec=pltpu.PrefetchScalarGridSpec(
            num_scalar_prefetch=2,
            grid=(M_TILES,),
            in_specs=[
                pl.BlockSpec((TM, D),
                             lambda m, te, tv: (_sel_f(tv[m], m), 0)),
                pl.BlockSpec((1, D, F), lambda m, te, tv: (te[m], 0, 0)),
                pl.BlockSpec((1, 1, F), lambda m, te, tv: (te[m], 0, 0)),
                pl.BlockSpec((1, F, D), lambda m, te, tv: (te[m], 0, 0)),
                pl.BlockSpec((1, 1, D), lambda m, te, tv: (te[m], 0, 0)),
                pl.BlockSpec((TM, 1),
                             lambda m, te, tv: (_sel_f(tv[m], m), 0)),
            ],
            out_specs=pl.BlockSpec((TM, D), lambda m, te, tv: (m, 0)),
        ),
        out_shape=jax.ShapeDtypeStruct((PADDED, D), jnp.float32),
        compiler_params=pltpu.CompilerParams(
            dimension_semantics=("arbitrary",)),
    )(te, tv, xs, W1b, b1r, W2b, b2r, ss2)

    unsort = pl.kernel(
        _unsort_body,
        out_type=jax.ShapeDtypeStruct((T, D), jnp.float32),
        mesh=plsc.VectorSubcoreMesh(core_axis_name="c", subcore_axis_name="s"),
        scratch_types=(
            pltpu.VMEM((_GCHUNK,), jnp.int32),
            pltpu.VMEM((_GCHUNK, D), jnp.float32),
            pltpu.SemaphoreType.DMA,
        ),
    )
    out = unsort(pos, ys)
    return out.reshape(1, T, D)


# speedup vs baseline: 1.0522x; 1.0099x over previous
import functools

import jax
import jax.numpy as jnp
from jax.experimental import pallas as pl
from jax.experimental.pallas import tpu as pltpu
import jax.experimental.pallas.tpu_sc as plsc

T = 2048
D = 768
F = 3072
E = 8
EPS = 1e-6

TM = 256
NF = 4
TF = F // NF
M_TILES = T // TM + E
PADDED = M_TILES * TM

GW_PAD = 128

_NC = 2
_NS = 16
_LANES = 16
_CHUNK = T // _NS
_NV = _CHUNK // _LANES
_HALF = _CHUNK // 2
_GCHUNK = T // (_NC * _NS)


def _gate_body(x_ref, nw_ref, gw_ref, xn_ref, sco_ref, idx_ref):
    xb = x_ref[...]
    ms = jnp.mean(xb * xb, axis=1, keepdims=True)
    xn = xb * jax.lax.rsqrt(ms + EPS) * nw_ref[...]
    xn_ref[...] = xn
    logits = jnp.dot(xn.astype(jnp.bfloat16),
                     gw_ref[...].astype(jnp.bfloat16),
                     preferred_element_type=jnp.float32)
    col = jax.lax.broadcasted_iota(jnp.int32, logits.shape, 1)
    logits = jnp.where(col < E, logits, -1e30)
    mx = jnp.max(logits, axis=1, keepdims=True)
    idx_ref[...] = jnp.min(jnp.where(logits >= mx, col, E), axis=1,
                           keepdims=True)
    ssum = jnp.sum(jnp.exp(logits - mx), axis=1, keepdims=True)
    sco_ref[...] = 1.0 / ssum


def _route_body(idx_hbm, sco_hbm, xn_hbm,
                xs_hbm, ss_hbm, pos_hbm, te_hbm, tv_hbm,
                idx_v, sco_v, pos_v, pos_lo, pos_hi, rows_v,
                cnt_v, all_v, te_v, tv_v,
                counts_sh, sco_sh, sem):
    cid = jax.lax.axis_index("c")
    sid = jax.lax.axis_index("s")
    tok0 = sid * _CHUNK
    lane = jax.lax.iota(jnp.int32, _LANES)
    zeros = jnp.zeros((_LANES,), jnp.int32)
    ones = jnp.ones((_LANES,), jnp.int32)

    pltpu.sync_copy(idx_hbm.at[pl.ds(tok0, _CHUNK)], idx_v)
    cnt = zeros
    for v in range(_NV):
        xv = idx_v[pl.ds(v * _LANES, _LANES)]
        for e in range(E):
            c = plsc.all_reduce_population_count(xv == e)
            cnt = jnp.where(lane == e, cnt + c, cnt)
    cnt_v[...] = cnt
    pltpu.sync_copy(cnt_v, counts_sh.at[pl.ds(sid * _LANES, _LANES)])
    plsc.subcore_barrier()

    pltpu.sync_copy(counts_sh, all_v)
    tot = zeros
    pre = zeros
    svec = jnp.full((_LANES,), sid, jnp.int32)
    for t in range(_NS):
        ct = all_v[pl.ds(t * _LANES, _LANES)]
        tvec = jnp.full((_LANES,), t, jnp.int32)
        pre = jnp.where(tvec < svec, pre + ct, pre)
        tot = tot + ct
    al = ((tot + (TM - 1)) >> 8) << 8
    cs = plsc.cumsum(al)
    start = cs - al
    base = start + pre

    b = [jnp.full((_LANES,), base[e], jnp.int32) for e in range(E)]
    for v in range(_NV):
        xv = idx_v[pl.ds(v * _LANES, _LANES)]
        pos = zeros
        for e in range(E):
            m = xv == e
            incl = plsc.cumsum(jnp.where(m, ones, zeros))
            pos = jnp.where(m, b[e] + incl - 1, pos)
            b[e] = b[e] + plsc.all_reduce_population_count(m)
        pos_v[pl.ds(v * _LANES, _LANES)] = pos
        if v < _NV // 2:
            pos_lo[pl.ds(v * _LANES, _LANES)] = pos
        else:
            pos_hi[pl.ds((v - _NV // 2) * _LANES, _LANES)] = pos

    @pl.when(cid == 0)
    def _():
        pltpu.sync_copy(pos_v, pos_hbm.at[pl.ds(tok0, _CHUNK)])
        pltpu.sync_copy(xn_hbm.at[pl.ds(tok0, _HALF)], rows_v)
        pltpu.async_copy(rows_v, xs_hbm.at[pos_lo], sem).wait()

    @pl.when(cid == 1)
    def _():
        pltpu.sync_copy(sco_hbm.at[pl.ds(tok0, _CHUNK)], sco_v)
        pltpu.sync_copy(sco_v, sco_sh.at[pos_v])
        pltpu.sync_copy(xn_hbm.at[pl.ds(tok0 + _HALF, _HALF)], rows_v)
        pltpu.async_copy(rows_v, xs_hbm.at[pos_hi], sem).wait()

    plsc.subcore_barrier()

    @pl.when((cid == 1) & (sid == 0))
    def _():
        pltpu.sync_copy(sco_sh, ss_hbm)

    @pl.when((cid == 0) & (sid == 0))
    def _():
        e_last = zeros
        for e in range(E):
            e_tot = jnp.full((_LANES,), tot[e], jnp.int32)
            e_vec = jnp.full((_LANES,), e, jnp.int32)
            e_last = jnp.where(e_tot > 0, e_vec, e_last)
        m16 = lane * TM
        te = e_last
        tv = zeros
        for e in range(E):
            s_e = jnp.full((_LANES,), start[e], jnp.int32)
            a_e = jnp.full((_LANES,), al[e], jnp.int32)
            in_r = (m16 >= s_e) & (m16 < s_e + a_e)
            te = jnp.where(in_r, jnp.full((_LANES,), e, jnp.int32), te)
            tv = jnp.where(in_r, ones, tv)
        te_v[...] = te
        tv_v[...] = tv
        pltpu.sync_copy(te_v, te_hbm)
        pltpu.sync_copy(tv_v, tv_hbm)


def _cast_body(w1_ref, w2_ref, o1_ref, o2_ref):
    o1_ref[...] = w1_ref[...].astype(jnp.bfloat16)
    o2_ref[...] = w2_ref[...].astype(jnp.bfloat16)


def _ffn_body(te_ref, tv_ref, x_ref, w1_ref, b1_ref, w2_ref, b2_ref, ss_ref,
              out_ref):
    m = pl.program_id(0)

    @pl.when(tv_ref[m] != 0)
    def _():
        h = jnp.dot(x_ref[...].astype(jnp.bfloat16), w1_ref[0],
                    preferred_element_type=jnp.float32) + b1_ref[0]
        h = h * (1.0 / (1.0 + jnp.exp(-h)))
        y = jnp.dot(h.astype(jnp.bfloat16), w2_ref[0],
                    preferred_element_type=jnp.float32)
        out_ref[...] = (y + b2_ref[0]) * ss_ref[...]


def _unsort_body(pos_hbm, ys_hbm, out_hbm, pidx, rows, sem):
    cid = jax.lax.axis_index("c")
    sid = jax.lax.axis_index("s")
    t0 = (sid * _NC + cid) * _GCHUNK
    pltpu.sync_copy(pos_hbm.at[pl.ds(t0, _GCHUNK)], pidx)
    pltpu.async_copy(ys_hbm.at[pidx], rows, sem).wait()
    pltpu.sync_copy(rows, out_hbm.at[pl.ds(t0, _GCHUNK)])


def _sel_f(tv, f):
    return jnp.where(tv != 0, f, 0)


@jax.jit
def kernel(x, norm_w, gate_w, W1, b1, W2, b2):
    x2 = x.reshape(T, D)
    nw2 = norm_w.reshape(1, D)
    gw_p = jnp.zeros((D, GW_PAD), jnp.float32).at[:, :E].set(gate_w)

    xn, sco2, idx2 = pl.pallas_call(
        _gate_body,
        grid=(T // TM,),
        in_specs=[
            pl.BlockSpec((TM, D), lambda m: (m, 0)),
            pl.BlockSpec((1, D), lambda m: (0, 0)),
            pl.BlockSpec((D, GW_PAD), lambda m: (0, 0)),
        ],
        out_specs=[
            pl.BlockSpec((TM, D), lambda m: (m, 0)),
            pl.BlockSpec((TM, 1), lambda m: (m, 0)),
            pl.BlockSpec((TM, 1), lambda m: (m, 0)),
        ],
        out_shape=[
            jax.ShapeDtypeStruct((T, D), jnp.float32),
            jax.ShapeDtypeStruct((T, 1), jnp.float32),
            jax.ShapeDtypeStruct((T, 1), jnp.int32),
        ],
        compiler_params=pltpu.CompilerParams(
            dimension_semantics=("arbitrary",)),
    )(x2, nw2, gw_p)
    idx1 = idx2.reshape(T)
    sco1 = sco2.reshape(T)

    mesh = plsc.VectorSubcoreMesh(core_axis_name="c", subcore_axis_name="s")
    route = pl.kernel(
        _route_body,
        out_type=(
            jax.ShapeDtypeStruct((PADDED, D), jnp.float32),
            jax.ShapeDtypeStruct((PADDED,), jnp.float32),
            jax.ShapeDtypeStruct((T,), jnp.int32),
            jax.ShapeDtypeStruct((_LANES,), jnp.int32),
            jax.ShapeDtypeStruct((_LANES,), jnp.int32),
        ),
        mesh=mesh,
        scratch_types=(
            pltpu.VMEM((_CHUNK,), jnp.int32),
            pltpu.VMEM((_CHUNK,), jnp.float32),
            pltpu.VMEM((_CHUNK,), jnp.int32),
            pltpu.VMEM((_HALF,), jnp.int32),
            pltpu.VMEM((_HALF,), jnp.int32),
            pltpu.VMEM((_HALF, D), jnp.float32),
            pltpu.VMEM((_LANES,), jnp.int32),
            pltpu.VMEM((_NS * _LANES,), jnp.int32),
            pltpu.VMEM((_LANES,), jnp.int32),
            pltpu.VMEM((_LANES,), jnp.int32),
            pltpu.VMEM_SHARED((_NS * _LANES,), jnp.int32),
            pltpu.VMEM_SHARED((PADDED,), jnp.float32),
            pltpu.SemaphoreType.DMA,
        ),
        compiler_params=pltpu.CompilerParams(needs_layout_passes=False),
    )
    xs, ss, pos, te, tv = route(idx1, sco1, xn)
    ss2 = ss.reshape(PADDED, 1)
    b1r = b1.reshape(E, 1, F)
    b2r = b2.reshape(E, 1, D)
    W1b, W2b = pl.pallas_call(
        _cast_body,
        grid=(E, 2),
        in_specs=[
            pl.BlockSpec((1, D, F // 2), lambda e, i: (e, 0, i)),
            pl.BlockSpec((1, F // 2, D), lambda e, i: (e, i, 0)),
        ],
        out_specs=[
            pl.BlockSpec((1, D, F // 2), lambda e, i: (e, 0, i)),
            pl.BlockSpec((1, F // 2, D), lambda e, i: (e, i, 0)),
        ],
        out_shape=[
            jax.ShapeDtypeStruct((E, D, F), jnp.bfloat16),
            jax.ShapeDtypeStruct((E, F, D), jnp.bfloat16),
        ],
        compiler_params=pltpu.CompilerParams(
            dimension_semantics=("arbitrary", "arbitrary")),
    )(W1, W2)

    ys = pl.pallas_call(
        _ffn_body,
        grid_spec=pltpu.PrefetchScalarGridSpec(
            num_scalar_prefetch=2,
            grid=(M_TILES,),
            in_specs=[
                pl.BlockSpec((TM, D),
                             lambda m, te, tv: (_sel_f(tv[m], m), 0)),
                pl.BlockSpec((1, D, F), lambda m, te, tv: (te[m], 0, 0)),
                pl.BlockSpec((1, 1, F), lambda m, te, tv: (te[m], 0, 0)),
                pl.BlockSpec((1, F, D), lambda m, te, tv: (te[m], 0, 0)),
                pl.BlockSpec((1, 1, D), lambda m, te, tv: (te[m], 0, 0)),
                pl.BlockSpec((TM, 1),
                             lambda m, te, tv: (_sel_f(tv[m], m), 0)),
            ],
            out_specs=pl.BlockSpec((TM, D), lambda m, te, tv: (m, 0)),
        ),
        out_shape=jax.ShapeDtypeStruct((PADDED, D), jnp.float32),
        compiler_params=pltpu.CompilerParams(
            dimension_semantics=("arbitrary",)),
    )(te, tv, xs, W1b, b1r, W2b, b2r, ss2)

    unsort = pl.kernel(
        _unsort_body,
        out_type=jax.ShapeDtypeStruct((T, D), jnp.float32),
        mesh=plsc.VectorSubcoreMesh(core_axis_name="c", subcore_axis_name="s"),
        scratch_types=(
            pltpu.VMEM((_GCHUNK,), jnp.int32),
            pltpu.VMEM((_GCHUNK, D), jnp.float32),
            pltpu.SemaphoreType.DMA,
        ),
    )
    out = unsort(pos, ys)
    return out.reshape(1, T, D)


# speedup vs baseline: 1.3314x; 1.2653x over previous
import functools

import jax
import jax.numpy as jnp
from jax.experimental import pallas as pl
from jax.experimental.pallas import tpu as pltpu
import jax.experimental.pallas.tpu_sc as plsc

T = 2048
D = 768
F = 3072
E = 8
EPS = 1e-6

TM = 256
NF = 4
TF = F // NF
M_TILES = T // TM + E
PADDED = M_TILES * TM

GW_PAD = 128

_NC = 2
_NS = 16
_LANES = 16
_CHUNK = T // _NS
_NV = _CHUNK // _LANES
_HALF = _CHUNK // 2
_GCHUNK = T // (_NC * _NS)


def _gate_body(x_ref, nw_ref, gw_ref, xn_ref, sco_ref, idx_ref):
    xb = x_ref[...]
    ms = jnp.mean(xb * xb, axis=1, keepdims=True)
    xn = xb * jax.lax.rsqrt(ms + EPS) * nw_ref[...]
    xn_ref[...] = xn
    logits = jnp.dot(xn.astype(jnp.bfloat16),
                     gw_ref[...].astype(jnp.bfloat16),
                     preferred_element_type=jnp.float32)
    col = jax.lax.broadcasted_iota(jnp.int32, logits.shape, 1)
    logits = jnp.where(col < E, logits, -1e30)
    mx = jnp.max(logits, axis=1, keepdims=True)
    idx_ref[...] = jnp.min(jnp.where(logits >= mx, col, E), axis=1,
                           keepdims=True)
    ssum = jnp.sum(jnp.exp(logits - mx), axis=1, keepdims=True)
    sco_ref[...] = 1.0 / ssum


def _route_body(idx_hbm, sco_hbm, xn_hbm,
                xs_hbm, ss_hbm, pos_hbm, te_hbm, tv_hbm,
                idx_v, sco_v, pos_v, pos_lo, pos_hi, rows_v,
                cnt_v, all_v, te_v, tv_v,
                counts_sh, sco_sh, sem):
    cid = jax.lax.axis_index("c")
    sid = jax.lax.axis_index("s")
    tok0 = sid * _CHUNK
    lane = jax.lax.iota(jnp.int32, _LANES)
    zeros = jnp.zeros((_LANES,), jnp.int32)
    ones = jnp.ones((_LANES,), jnp.int32)

    pltpu.sync_copy(idx_hbm.at[pl.ds(tok0, _CHUNK)], idx_v)
    cnt = zeros
    for v in range(_NV):
        xv = idx_v[pl.ds(v * _LANES, _LANES)]
        for e in range(E):
            c = plsc.all_reduce_population_count(xv == e)
            cnt = jnp.where(lane == e, cnt + c, cnt)
    cnt_v[...] = cnt
    pltpu.sync_copy(cnt_v, counts_sh.at[pl.ds(sid * _LANES, _LANES)])
    plsc.subcore_barrier()

    pltpu.sync_copy(counts_sh, all_v)
    tot = zeros
    pre = zeros
    svec = jnp.full((_LANES,), sid, jnp.int32)
    for t in range(_NS):
        ct = all_v[pl.ds(t * _LANES, _LANES)]
        tvec = jnp.full((_LANES,), t, jnp.int32)
        pre = jnp.where(tvec < svec, pre + ct, pre)
        tot = tot + ct
    al = ((tot + (TM - 1)) >> 8) << 8
    cs = plsc.cumsum(al)
    start = cs - al
    base = start + pre

    b = [jnp.full((_LANES,), base[e], jnp.int32) for e in range(E)]
    for v in range(_NV):
        xv = idx_v[pl.ds(v * _LANES, _LANES)]
        pos = zeros
        for e in range(E):
            m = xv == e
            incl = plsc.cumsum(jnp.where(m, ones, zeros))
            pos = jnp.where(m, b[e] + incl - 1, pos)
            b[e] = b[e] + plsc.all_reduce_population_count(m)
        pos_v[pl.ds(v * _LANES, _LANES)] = pos
        if v < _NV // 2:
            pos_lo[pl.ds(v * _LANES, _LANES)] = pos
        else:
            pos_hi[pl.ds((v - _NV // 2) * _LANES, _LANES)] = pos

    @pl.when(cid == 0)
    def _():
        pltpu.sync_copy(pos_v, pos_hbm.at[pl.ds(tok0, _CHUNK)])
        pltpu.sync_copy(xn_hbm.at[pl.ds(tok0, _HALF)], rows_v)
        pltpu.async_copy(rows_v, xs_hbm.at[pos_lo], sem).wait()

    @pl.when(cid == 1)
    def _():
        pltpu.sync_copy(sco_hbm.at[pl.ds(tok0, _CHUNK)], sco_v)
        pltpu.sync_copy(sco_v, sco_sh.at[pos_v])
        pltpu.sync_copy(xn_hbm.at[pl.ds(tok0 + _HALF, _HALF)], rows_v)
        pltpu.async_copy(rows_v, xs_hbm.at[pos_hi], sem).wait()

    plsc.subcore_barrier()

    @pl.when((cid == 1) & (sid == 0))
    def _():
        pltpu.sync_copy(sco_sh, ss_hbm)

    @pl.when((cid == 0) & (sid == 0))
    def _():
        e_last = zeros
        for e in range(E):
            e_tot = jnp.full((_LANES,), tot[e], jnp.int32)
            e_vec = jnp.full((_LANES,), e, jnp.int32)
            e_last = jnp.where(e_tot > 0, e_vec, e_last)
        m16 = lane * TM
        te = e_last
        tv = zeros
        for e in range(E):
            s_e = jnp.full((_LANES,), start[e], jnp.int32)
            a_e = jnp.full((_LANES,), al[e], jnp.int32)
            in_r = (m16 >= s_e) & (m16 < s_e + a_e)
            te = jnp.where(in_r, jnp.full((_LANES,), e, jnp.int32), te)
            tv = jnp.where(in_r, ones, tv)
        te_v[...] = te
        tv_v[...] = tv
        pltpu.sync_copy(te_v, te_hbm)
        pltpu.sync_copy(tv_v, tv_hbm)


def _ffn_body(te_ref, tv_ref, x_ref, w1_ref, b1_ref, w2_ref, b2_ref, ss_ref,
              out_ref, w1b, w2b):
    m = pl.program_id(0)

    @pl.when(tv_ref[m] != 0)
    def _():
        recast = jnp.logical_or(m == 0,
                                te_ref[m] != te_ref[jnp.maximum(m - 1, 0)])

        @pl.when(recast)
        def _():
            w1b[...] = w1_ref[0].astype(jnp.bfloat16)
            w2b[...] = w2_ref[0].astype(jnp.bfloat16)

        h = jnp.dot(x_ref[...].astype(jnp.bfloat16), w1b[...],
                    preferred_element_type=jnp.float32) + b1_ref[0]
        h = h * (1.0 / (1.0 + jnp.exp(-h)))
        y = jnp.dot(h.astype(jnp.bfloat16), w2b[...],
                    preferred_element_type=jnp.float32)
        out_ref[...] = (y + b2_ref[0]) * ss_ref[...]


def _unsort_body(pos_hbm, ys_hbm, out_hbm, pidx, rows, sem):
    cid = jax.lax.axis_index("c")
    sid = jax.lax.axis_index("s")
    t0 = (sid * _NC + cid) * _GCHUNK
    pltpu.sync_copy(pos_hbm.at[pl.ds(t0, _GCHUNK)], pidx)
    pltpu.async_copy(ys_hbm.at[pidx], rows, sem).wait()
    pltpu.sync_copy(rows, out_hbm.at[pl.ds(t0, _GCHUNK)])


def _sel_f(tv, f):
    return jnp.where(tv != 0, f, 0)


@jax.jit
def kernel(x, norm_w, gate_w, W1, b1, W2, b2):
    x2 = x.reshape(T, D)
    nw2 = norm_w.reshape(1, D)
    gw_p = jnp.zeros((D, GW_PAD), jnp.float32).at[:, :E].set(gate_w)

    xn, sco2, idx2 = pl.pallas_call(
        _gate_body,
        grid=(T // TM,),
        in_specs=[
            pl.BlockSpec((TM, D), lambda m: (m, 0)),
            pl.BlockSpec((1, D), lambda m: (0, 0)),
            pl.BlockSpec((D, GW_PAD), lambda m: (0, 0)),
        ],
        out_specs=[
            pl.BlockSpec((TM, D), lambda m: (m, 0)),
            pl.BlockSpec((TM, 1), lambda m: (m, 0)),
            pl.BlockSpec((TM, 1), lambda m: (m, 0)),
        ],
        out_shape=[
            jax.ShapeDtypeStruct((T, D), jnp.float32),
            jax.ShapeDtypeStruct((T, 1), jnp.float32),
            jax.ShapeDtypeStruct((T, 1), jnp.int32),
        ],
        compiler_params=pltpu.CompilerParams(
            dimension_semantics=("arbitrary",)),
    )(x2, nw2, gw_p)
    idx1 = idx2.reshape(T)
    sco1 = sco2.reshape(T)

    mesh = plsc.VectorSubcoreMesh(core_axis_name="c", subcore_axis_name="s")
    route = pl.kernel(
        _route_body,
        out_type=(
            jax.ShapeDtypeStruct((PADDED, D), jnp.float32),
            jax.ShapeDtypeStruct((PADDED,), jnp.float32),
            jax.ShapeDtypeStruct((T,), jnp.int32),
            jax.ShapeDtypeStruct((_LANES,), jnp.int32),
            jax.ShapeDtypeStruct((_LANES,), jnp.int32),
        ),
        mesh=mesh,
        scratch_types=(
            pltpu.VMEM((_CHUNK,), jnp.int32),
            pltpu.VMEM((_CHUNK,), jnp.float32),
            pltpu.VMEM((_CHUNK,), jnp.int32),
            pltpu.VMEM((_HALF,), jnp.int32),
            pltpu.VMEM((_HALF,), jnp.int32),
            pltpu.VMEM((_HALF, D), jnp.float32),
            pltpu.VMEM((_LANES,), jnp.int32),
            pltpu.VMEM((_NS * _LANES,), jnp.int32),
            pltpu.VMEM((_LANES,), jnp.int32),
            pltpu.VMEM((_LANES,), jnp.int32),
            pltpu.VMEM_SHARED((_NS * _LANES,), jnp.int32),
            pltpu.VMEM_SHARED((PADDED,), jnp.float32),
            pltpu.SemaphoreType.DMA,
        ),
        compiler_params=pltpu.CompilerParams(needs_layout_passes=False),
    )
    xs, ss, pos, te, tv = route(idx1, sco1, xn)
    ss2 = ss.reshape(PADDED, 1)
    b1r = b1.reshape(E, 1, F)
    b2r = b2.reshape(E, 1, D)

    ys = pl.pallas_call(
        _ffn_body,
        grid_spec=pltpu.PrefetchScalarGridSpec(
            num_scalar_prefetch=2,
            grid=(M_TILES,),
            in_specs=[
                pl.BlockSpec((TM, D),
                             lambda m, te, tv: (_sel_f(tv[m], m), 0)),
                pl.BlockSpec((1, D, F), lambda m, te, tv: (te[m], 0, 0)),
                pl.BlockSpec((1, 1, F), lambda m, te, tv: (te[m], 0, 0)),
                pl.BlockSpec((1, F, D), lambda m, te, tv: (te[m], 0, 0)),
                pl.BlockSpec((1, 1, D), lambda m, te, tv: (te[m], 0, 0)),
                pl.BlockSpec((TM, 1),
                             lambda m, te, tv: (_sel_f(tv[m], m), 0)),
            ],
            out_specs=pl.BlockSpec((TM, D), lambda m, te, tv: (m, 0)),
            scratch_shapes=[
                pltpu.VMEM((D, F), jnp.bfloat16),
                pltpu.VMEM((F, D), jnp.bfloat16),
            ],
        ),
        out_shape=jax.ShapeDtypeStruct((PADDED, D), jnp.float32),
        compiler_params=pltpu.CompilerParams(
            dimension_semantics=("arbitrary",)),
    )(te, tv, xs, W1, b1r, W2, b2r, ss2)

    unsort = pl.kernel(
        _unsort_body,
        out_type=jax.ShapeDtypeStruct((T, D), jnp.float32),
        mesh=plsc.VectorSubcoreMesh(core_axis_name="c", subcore_axis_name="s"),
        scratch_types=(
            pltpu.VMEM((_GCHUNK,), jnp.int32),
            pltpu.VMEM((_GCHUNK, D), jnp.float32),
            pltpu.SemaphoreType.DMA,
        ),
    )
    out = unsort(pos, ys)
    return out.reshape(1, T, D)
